# baseline (speedup 1.0000x reference)
import jax
import jax.numpy as jnp
from jax.experimental import pallas as pl

_B, _N = 8, 1024
_D = 6
_E = 3
_K = 6


def _body(x_ref, xT_ref, We1T_ref, be1_ref, We2T_ref, be2_ref, WgT_ref, bg_ref,
          Wc1T_ref, bc1_ref, Wc2T_ref, bc2_ref, ln_g_ref, ln_b_ref,
          Wn1T_ref, bn1_ref, Wn2T_ref, bn2_ref, Wm1T_ref, bm1_ref, Wm2T_ref, bm2_ref,
          out_ref):
    N = _N
    xb = x_ref[0]
    coors = xb[:, _D:_D + _E]
    ct = xT_ref[0]
    fT = ct[:_D]
    cT = ct[_D:_D + _E]

    dx = coors[:, 0:1] - ct[_D + 0:_D + 1, :]
    dy = coors[:, 1:2] - ct[_D + 1:_D + 2, :]
    dz = coors[:, 2:3] - ct[_D + 2:_D + 3, :]
    dist = dx * dx + dy * dy + dz * dz

    fj = jax.lax.broadcasted_iota(jnp.int32, (N, N), 0)
    gTs, rds = [], []
    for _ in range(_K):
        m = jnp.min(dist, axis=0, keepdims=True)
        cand = jnp.where(dist <= m, fj, N)
        idx = jnp.min(cand, axis=0, keepdims=True)
        onehot = fj == idx
        gT = jnp.dot(ct, jnp.where(onehot, 1.0, 0.0),
                     preferred_element_type=jnp.float32,
                     precision=jax.lax.Precision.HIGHEST)
        dist = jnp.where(onehot, jnp.inf, dist)
        gTs.append(gT)
        rds.append(m)

    fjT = jnp.concatenate([g[:_D] for g in gTs], axis=1)
    cjT = jnp.concatenate([g[_D:_D + _E] for g in gTs], axis=1)
    rdT = jnp.concatenate(rds, axis=1)
    fiT = jnp.concatenate([fT] * _K, axis=1)
    ciT = jnp.concatenate([cT] * _K, axis=1)
    relT = ciT - cjT

    edge_inT = jnp.concatenate([fiT, fjT, rdT], axis=0)
    hT = jax.nn.silu(jnp.dot(We1T_ref[...], edge_inT,
                             preferred_element_type=jnp.float32) + be1_ref[...])
    m_ijT = jax.nn.silu(jnp.dot(We2T_ref[...], hT,
                                preferred_element_type=jnp.float32) + be2_ref[...])
    gateT = jax.nn.sigmoid(jnp.dot(WgT_ref[...], m_ijT,
                                   preferred_element_type=jnp.float32) + bg_ref[...])
    m_ijT = m_ijT * gateT
    cwT = (jnp.dot(Wc2T_ref[...],
                   jax.nn.silu(jnp.dot(Wc1T_ref[...], m_ijT,
                                       preferred_element_type=jnp.float32) + bc1_ref[...]),
                   preferred_element_type=jnp.float32) + bc2_ref[...])
    cwT = jnp.clip(cwT, -1.0, 1.0)

    normT = jnp.sqrt(relT[0:1] ** 2 + relT[1:2] ** 2 + relT[2:3] ** 2)
    relnT = relT / jnp.maximum(normT, 1e-8)
    contribT = cwT * relnT
    coorsoT = cT + sum(contribT[:, k * N:(k + 1) * N] for k in range(_K))
    m_iT = sum(m_ijT[:, k * N:(k + 1) * N] for k in range(_K))

    mu = fT.mean(axis=0, keepdims=True)
    var = ((fT - mu) ** 2).mean(axis=0, keepdims=True)
    nfT = (fT - mu) / jnp.sqrt(var + 1e-5) * ln_g_ref[...] + ln_b_ref[...]
    node_inT = jnp.concatenate([nfT, m_iT], axis=0)
    nodeoT = (jnp.dot(Wn2T_ref[...],
                      jax.nn.silu(jnp.dot(Wn1T_ref[...], node_inT,
                                          preferred_element_type=jnp.float32) + bn1_ref[...]),
                      preferred_element_type=jnp.float32)
              + bn2_ref[...] + fT)

    zT = jnp.concatenate([nodeoT, coorsoT], axis=0)
    zmT = jnp.mean(zT, axis=1, keepdims=True)
    zz = (jnp.dot(Wm2T_ref[...],
                  jax.nn.relu(jnp.dot(Wm1T_ref[...], zmT,
                                      preferred_element_type=jnp.float32) + bm1_ref[...]),
                  preferred_element_type=jnp.float32) + bm2_ref[...])
    out_ref[0] = zz


def kernel(x, We1, be1, We2, be2, Wg, bg, Wc1, bc1, Wc2, bc2, ln_g, ln_b,
           Wn1, bn1, Wn2, bn2, Wm1, bm1, Wm2, bm2, interpret=False):
    xT = jnp.swapaxes(x, 1, 2)
    col = lambda a: a.reshape(-1, 1)
    full = lambda shp: pl.BlockSpec(shp, lambda b: (0,) * len(shp))
    wspec = lambda a: full(a.shape)
    args = [x, xT,
            We1.T, col(be1), We2.T, col(be2), Wg.T, col(bg),
            Wc1.T, col(bc1), Wc2.T, col(bc2), col(ln_g), col(ln_b),
            Wn1.T, col(bn1), Wn2.T, col(bn2), Wm1.T, col(bm1), Wm2.T, col(bm2)]
    out = pl.pallas_call(
        _body,
        grid=(_B,),
        in_specs=[pl.BlockSpec((1, _N, _D + _E), lambda b: (b, 0, 0)),
                  pl.BlockSpec((1, _D + _E, _N), lambda b: (b, 0, 0))]
                 + [wspec(a) for a in args[2:]],
        out_specs=pl.BlockSpec((1, 36, 1), lambda b: (b, 0, 0)),
        out_shape=jax.ShapeDtypeStruct((_B, 36, 1), jnp.float32),
        interpret=interpret,
    )(*args)
    z = out.reshape(_B, 2, 18)
    return jnp.pad(z, ((0, 0), (0, 27), (0, 0)))


# speedup vs baseline: 1.1229x; 1.1229x over previous
import jax
import jax.numpy as jnp
from jax.experimental import pallas as pl

_B, _N = 8, 1024
_D = 6
_E = 3
_K = 6


def _body(x_ref, xT_ref, We1T_ref, be1_ref, We2T_ref, be2_ref, WgT_ref, bg_ref,
          Wc1T_ref, bc1_ref, Wc2T_ref, bc2_ref, ln_g_ref, ln_b_ref,
          Wn1T_ref, bn1_ref, Wn2T_ref, bn2_ref, Wm1T_ref, bm1_ref, Wm2T_ref, bm2_ref,
          out_ref):
    N = _N
    xb = x_ref[0]
    coors = xb[:, _D:_D + _E]
    ct = xT_ref[0]
    fT = ct[:_D]
    cT = ct[_D:_D + _E]

    dx = coors[:, 0:1] - ct[_D + 0:_D + 1, :]
    dy = coors[:, 1:2] - ct[_D + 1:_D + 2, :]
    dz = coors[:, 2:3] - ct[_D + 2:_D + 3, :]
    dist = dx * dx + dy * dy + dz * dz

    fj = jax.lax.broadcasted_iota(jnp.int32, (N, N), 0)
    fi = jax.lax.broadcasted_iota(jnp.int32, (N, N), 1)
    gTs = [ct]
    rds = [jnp.zeros((1, N), jnp.float32)]
    dist = jnp.where(fj == fi, jnp.inf, dist)
    for _ in range(_K - 1):
        m = jnp.min(dist, axis=0, keepdims=True)
        cand = jnp.where(dist <= m, fj, N)
        idx = jnp.min(cand, axis=0, keepdims=True)
        onehot = fj == idx
        gT = jnp.dot(ct, jnp.where(onehot, 1.0, 0.0),
                     preferred_element_type=jnp.float32,
                     precision=jax.lax.Precision.HIGHEST)
        dist = jnp.where(onehot, jnp.inf, dist)
        gTs.append(gT)
        rds.append(m)

    fjT = jnp.concatenate([g[:_D] for g in gTs], axis=1)
    cjT = jnp.concatenate([g[_D:_D + _E] for g in gTs], axis=1)
    rdT = jnp.concatenate(rds, axis=1)
    fiT = jnp.concatenate([fT] * _K, axis=1)
    ciT = jnp.concatenate([cT] * _K, axis=1)
    relT = ciT - cjT

    edge_inT = jnp.concatenate([fiT, fjT, rdT], axis=0)
    hT = jax.nn.silu(jnp.dot(We1T_ref[...], edge_inT,
                             preferred_element_type=jnp.float32) + be1_ref[...])
    m_ijT = jax.nn.silu(jnp.dot(We2T_ref[...], hT,
                                preferred_element_type=jnp.float32) + be2_ref[...])
    gateT = jax.nn.sigmoid(jnp.dot(WgT_ref[...], m_ijT,
                                   preferred_element_type=jnp.float32) + bg_ref[...])
    m_ijT = m_ijT * gateT
    cwT = (jnp.dot(Wc2T_ref[...],
                   jax.nn.silu(jnp.dot(Wc1T_ref[...], m_ijT,
                                       preferred_element_type=jnp.float32) + bc1_ref[...]),
                   preferred_element_type=jnp.float32) + bc2_ref[...])
    cwT = jnp.clip(cwT, -1.0, 1.0)

    normT = jnp.sqrt(relT[0:1] ** 2 + relT[1:2] ** 2 + relT[2:3] ** 2)
    relnT = relT / jnp.maximum(normT, 1e-8)
    contribT = cwT * relnT
    coorsoT = cT + sum(contribT[:, k * N:(k + 1) * N] for k in range(_K))
    m_iT = sum(m_ijT[:, k * N:(k + 1) * N] for k in range(_K))

    mu = fT.mean(axis=0, keepdims=True)
    var = ((fT - mu) ** 2).mean(axis=0, keepdims=True)
    nfT = (fT - mu) / jnp.sqrt(var + 1e-5) * ln_g_ref[...] + ln_b_ref[...]
    node_inT = jnp.concatenate([nfT, m_iT], axis=0)
    nodeoT = (jnp.dot(Wn2T_ref[...],
                      jax.nn.silu(jnp.dot(Wn1T_ref[...], node_inT,
                                          preferred_element_type=jnp.float32) + bn1_ref[...]),
                      preferred_element_type=jnp.float32)
              + bn2_ref[...] + fT)

    zT = jnp.concatenate([nodeoT, coorsoT], axis=0)
    zmT = jnp.mean(zT, axis=1, keepdims=True)
    zz = (jnp.dot(Wm2T_ref[...],
                  jax.nn.relu(jnp.dot(Wm1T_ref[...], zmT,
                                      preferred_element_type=jnp.float32) + bm1_ref[...]),
                  preferred_element_type=jnp.float32) + bm2_ref[...])
    out_ref[0] = zz


def kernel(x, We1, be1, We2, be2, Wg, bg, Wc1, bc1, Wc2, bc2, ln_g, ln_b,
           Wn1, bn1, Wn2, bn2, Wm1, bm1, Wm2, bm2, interpret=False):
    xT = jnp.swapaxes(x, 1, 2)
    col = lambda a: a.reshape(-1, 1)
    full = lambda shp: pl.BlockSpec(shp, lambda b: (0,) * len(shp))
    wspec = lambda a: full(a.shape)
    args = [x, xT,
            We1.T, col(be1), We2.T, col(be2), Wg.T, col(bg),
            Wc1.T, col(bc1), Wc2.T, col(bc2), col(ln_g), col(ln_b),
            Wn1.T, col(bn1), Wn2.T, col(bn2), Wm1.T, col(bm1), Wm2.T, col(bm2)]
    out = pl.pallas_call(
        _body,
        grid=(_B,),
        in_specs=[pl.BlockSpec((1, _N, _D + _E), lambda b: (b, 0, 0)),
                  pl.BlockSpec((1, _D + _E, _N), lambda b: (b, 0, 0))]
                 + [wspec(a) for a in args[2:]],
        out_specs=pl.BlockSpec((1, 36, 1), lambda b: (b, 0, 0)),
        out_shape=jax.ShapeDtypeStruct((_B, 36, 1), jnp.float32),
        interpret=interpret,
    )(*args)
    z = out.reshape(_B, 2, 18)
    return jnp.pad(z, ((0, 0), (0, 27), (0, 0)))


# speedup vs baseline: 1.5726x; 1.4005x over previous
import jax
import jax.numpy as jnp
from jax.experimental import pallas as pl

_B, _N = 8, 1024
_D = 6
_E = 3
_K = 6


def _body(x_ref, xT_ref, We1T_ref, be1_ref, We2T_ref, be2_ref, WgT_ref, bg_ref,
          Wc1T_ref, bc1_ref, Wc2T_ref, bc2_ref, ln_g_ref, ln_b_ref,
          Wn1T_ref, bn1_ref, Wn2T_ref, bn2_ref, Wm1T_ref, bm1_ref, Wm2T_ref, bm2_ref,
          out_ref):
    N = _N
    xb = x_ref[0]
    coors = xb[:, _D:_D + _E]
    ct = xT_ref[0]
    fT = ct[:_D]
    cT = ct[_D:_D + _E]

    dx = coors[:, 0:1] - ct[_D + 0:_D + 1, :]
    dy = coors[:, 1:2] - ct[_D + 1:_D + 2, :]
    dz = coors[:, 2:3] - ct[_D + 2:_D + 3, :]
    dist = dx * dx + dy * dy + dz * dz

    fj = jax.lax.broadcasted_iota(jnp.int32, (N, N), 0)
    fi = jax.lax.broadcasted_iota(jnp.int32, (N, N), 1)
    ct_hi = ct.astype(jnp.bfloat16).astype(jnp.float32)
    r1 = ct - ct_hi
    ct_md = r1.astype(jnp.bfloat16).astype(jnp.float32)
    ct_lo = r1 - ct_md
    gTs = [ct]
    rds = [jnp.zeros((1, N), jnp.float32)]
    dist = jnp.where(fj == fi, jnp.inf, dist)
    for _ in range(_K - 1):
        m = jnp.min(dist, axis=0, keepdims=True)
        cand = jnp.where(dist <= m, fj, N)
        idx = jnp.min(cand, axis=0, keepdims=True)
        onehot = fj == idx
        oh = jnp.where(onehot, 1.0, 0.0)
        gT = ((jnp.dot(ct_lo, oh, preferred_element_type=jnp.float32)
               + jnp.dot(ct_md, oh, preferred_element_type=jnp.float32))
              + jnp.dot(ct_hi, oh, preferred_element_type=jnp.float32))
        dist = jnp.where(onehot, jnp.inf, dist)
        gTs.append(gT)
        rds.append(m)

    fjT = jnp.concatenate([g[:_D] for g in gTs], axis=1)
    cjT = jnp.concatenate([g[_D:_D + _E] for g in gTs], axis=1)
    rdT = jnp.concatenate(rds, axis=1)
    fiT = jnp.concatenate([fT] * _K, axis=1)
    ciT = jnp.concatenate([cT] * _K, axis=1)
    relT = ciT - cjT

    edge_inT = jnp.concatenate([fiT, fjT, rdT], axis=0)
    hT = jax.nn.silu(jnp.dot(We1T_ref[...], edge_inT,
                             preferred_element_type=jnp.float32) + be1_ref[...])
    m_ijT = jax.nn.silu(jnp.dot(We2T_ref[...], hT,
                                preferred_element_type=jnp.float32) + be2_ref[...])
    gateT = jax.nn.sigmoid(jnp.dot(WgT_ref[...], m_ijT,
                                   preferred_element_type=jnp.float32) + bg_ref[...])
    m_ijT = m_ijT * gateT
    cwT = (jnp.dot(Wc2T_ref[...],
                   jax.nn.silu(jnp.dot(Wc1T_ref[...], m_ijT,
                                       preferred_element_type=jnp.float32) + bc1_ref[...]),
                   preferred_element_type=jnp.float32) + bc2_ref[...])
    cwT = jnp.clip(cwT, -1.0, 1.0)

    normT = jnp.sqrt(relT[0:1] ** 2 + relT[1:2] ** 2 + relT[2:3] ** 2)
    relnT = relT / jnp.maximum(normT, 1e-8)
    contribT = cwT * relnT
    coorsoT = cT + sum(contribT[:, k * N:(k + 1) * N] for k in range(_K))
    m_iT = sum(m_ijT[:, k * N:(k + 1) * N] for k in range(_K))

    mu = fT.mean(axis=0, keepdims=True)
    var = ((fT - mu) ** 2).mean(axis=0, keepdims=True)
    nfT = (fT - mu) / jnp.sqrt(var + 1e-5) * ln_g_ref[...] + ln_b_ref[...]
    node_inT = jnp.concatenate([nfT, m_iT], axis=0)
    nodeoT = (jnp.dot(Wn2T_ref[...],
                      jax.nn.silu(jnp.dot(Wn1T_ref[...], node_inT,
                                          preferred_element_type=jnp.float32) + bn1_ref[...]),
                      preferred_element_type=jnp.float32)
              + bn2_ref[...] + fT)

    zT = jnp.concatenate([nodeoT, coorsoT], axis=0)
    zmT = jnp.mean(zT, axis=1, keepdims=True)
    zz = (jnp.dot(Wm2T_ref[...],
                  jax.nn.relu(jnp.dot(Wm1T_ref[...], zmT,
                                      preferred_element_type=jnp.float32) + bm1_ref[...]),
                  preferred_element_type=jnp.float32) + bm2_ref[...])
    out_ref[0] = zz


def kernel(x, We1, be1, We2, be2, Wg, bg, Wc1, bc1, Wc2, bc2, ln_g, ln_b,
           Wn1, bn1, Wn2, bn2, Wm1, bm1, Wm2, bm2, interpret=False):
    xT = jnp.swapaxes(x, 1, 2)
    col = lambda a: a.reshape(-1, 1)
    full = lambda shp: pl.BlockSpec(shp, lambda b: (0,) * len(shp))
    wspec = lambda a: full(a.shape)
    args = [x, xT,
            We1.T, col(be1), We2.T, col(be2), Wg.T, col(bg),
            Wc1.T, col(bc1), Wc2.T, col(bc2), col(ln_g), col(ln_b),
            Wn1.T, col(bn1), Wn2.T, col(bn2), Wm1.T, col(bm1), Wm2.T, col(bm2)]
    out = pl.pallas_call(
        _body,
        grid=(_B,),
        in_specs=[pl.BlockSpec((1, _N, _D + _E), lambda b: (b, 0, 0)),
                  pl.BlockSpec((1, _D + _E, _N), lambda b: (b, 0, 0))]
                 + [wspec(a) for a in args[2:]],
        out_specs=pl.BlockSpec((1, 36, 1), lambda b: (b, 0, 0)),
        out_shape=jax.ShapeDtypeStruct((_B, 36, 1), jnp.float32),
        interpret=interpret,
    )(*args)
    z = out.reshape(_B, 2, 18)
    return jnp.pad(z, ((0, 0), (0, 27), (0, 0)))


# speedup vs baseline: 1.7854x; 1.1353x over previous
import jax
import jax.numpy as jnp
from jax.experimental import pallas as pl

_B, _N = 8, 1024
_D = 6
_E = 3
_K = 6


def _body(x_ref, xT_ref, We1T_ref, be1_ref, We2T_ref, be2_ref, WgT_ref, bg_ref,
          Wc1T_ref, bc1_ref, Wc2T_ref, bc2_ref, ln_g_ref, ln_b_ref,
          Wn1T_ref, bn1_ref, Wn2T_ref, bn2_ref, Wm1T_ref, bm1_ref, Wm2T_ref, bm2_ref,
          out_ref):
    N = _N
    xb = x_ref[0]
    coors = xb[:, _D:_D + _E]
    ct = xT_ref[0]
    fT = ct[:_D]
    cT = ct[_D:_D + _E]

    dx = coors[:, 0:1] - ct[_D + 0:_D + 1, :]
    dy = coors[:, 1:2] - ct[_D + 1:_D + 2, :]
    dz = coors[:, 2:3] - ct[_D + 2:_D + 3, :]
    dist = dx * dx + dy * dy + dz * dz

    ct_hi = ct.astype(jnp.bfloat16).astype(jnp.float32)
    r1 = ct - ct_hi
    ct_md = r1.astype(jnp.bfloat16).astype(jnp.float32)
    ct_lo = r1 - ct_md
    gTs = [ct]
    rds = [jnp.zeros((1, N), jnp.float32)]
    m = jnp.zeros((1, N), jnp.float32)
    for _ in range(_K - 1):
        m = jnp.min(jnp.where(dist > m, dist, jnp.inf),
                    axis=0, keepdims=True)
        oh = jnp.where(dist == m, 1.0, 0.0)
        gT = ((jnp.dot(ct_lo, oh, preferred_element_type=jnp.float32)
               + jnp.dot(ct_md, oh, preferred_element_type=jnp.float32))
              + jnp.dot(ct_hi, oh, preferred_element_type=jnp.float32))
        gTs.append(gT)
        rds.append(m)

    fjT = jnp.concatenate([g[:_D] for g in gTs], axis=1)
    cjT = jnp.concatenate([g[_D:_D + _E] for g in gTs], axis=1)
    rdT = jnp.concatenate(rds, axis=1)
    fiT = jnp.concatenate([fT] * _K, axis=1)
    ciT = jnp.concatenate([cT] * _K, axis=1)
    relT = ciT - cjT

    edge_inT = jnp.concatenate([fiT, fjT, rdT], axis=0)
    hT = jax.nn.silu(jnp.dot(We1T_ref[...], edge_inT,
                             preferred_element_type=jnp.float32) + be1_ref[...])
    m_ijT = jax.nn.silu(jnp.dot(We2T_ref[...], hT,
                                preferred_element_type=jnp.float32) + be2_ref[...])
    gateT = jax.nn.sigmoid(jnp.dot(WgT_ref[...], m_ijT,
                                   preferred_element_type=jnp.float32) + bg_ref[...])
    m_ijT = m_ijT * gateT
    cwT = (jnp.dot(Wc2T_ref[...],
                   jax.nn.silu(jnp.dot(Wc1T_ref[...], m_ijT,
                                       preferred_element_type=jnp.float32) + bc1_ref[...]),
                   preferred_element_type=jnp.float32) + bc2_ref[...])
    cwT = jnp.clip(cwT, -1.0, 1.0)

    normT = jnp.sqrt(relT[0:1] ** 2 + relT[1:2] ** 2 + relT[2:3] ** 2)
    relnT = relT / jnp.maximum(normT, 1e-8)
    contribT = cwT * relnT
    coorsoT = cT + sum(contribT[:, k * N:(k + 1) * N] for k in range(_K))
    m_iT = sum(m_ijT[:, k * N:(k + 1) * N] for k in range(_K))

    mu = fT.mean(axis=0, keepdims=True)
    var = ((fT - mu) ** 2).mean(axis=0, keepdims=True)
    nfT = (fT - mu) / jnp.sqrt(var + 1e-5) * ln_g_ref[...] + ln_b_ref[...]
    node_inT = jnp.concatenate([nfT, m_iT], axis=0)
    nodeoT = (jnp.dot(Wn2T_ref[...],
                      jax.nn.silu(jnp.dot(Wn1T_ref[...], node_inT,
                                          preferred_element_type=jnp.float32) + bn1_ref[...]),
                      preferred_element_type=jnp.float32)
              + bn2_ref[...] + fT)

    zT = jnp.concatenate([nodeoT, coorsoT], axis=0)
    zmT = jnp.mean(zT, axis=1, keepdims=True)
    zz = (jnp.dot(Wm2T_ref[...],
                  jax.nn.relu(jnp.dot(Wm1T_ref[...], zmT,
                                      preferred_element_type=jnp.float32) + bm1_ref[...]),
                  preferred_element_type=jnp.float32) + bm2_ref[...])
    out_ref[0] = zz


def kernel(x, We1, be1, We2, be2, Wg, bg, Wc1, bc1, Wc2, bc2, ln_g, ln_b,
           Wn1, bn1, Wn2, bn2, Wm1, bm1, Wm2, bm2, interpret=False):
    xT = jnp.swapaxes(x, 1, 2)
    col = lambda a: a.reshape(-1, 1)
    full = lambda shp: pl.BlockSpec(shp, lambda b: (0,) * len(shp))
    wspec = lambda a: full(a.shape)
    args = [x, xT,
            We1.T, col(be1), We2.T, col(be2), Wg.T, col(bg),
            Wc1.T, col(bc1), Wc2.T, col(bc2), col(ln_g), col(ln_b),
            Wn1.T, col(bn1), Wn2.T, col(bn2), Wm1.T, col(bm1), Wm2.T, col(bm2)]
    out = pl.pallas_call(
        _body,
        grid=(_B,),
        in_specs=[pl.BlockSpec((1, _N, _D + _E), lambda b: (b, 0, 0)),
                  pl.BlockSpec((1, _D + _E, _N), lambda b: (b, 0, 0))]
                 + [wspec(a) for a in args[2:]],
        out_specs=pl.BlockSpec((1, 36, 1), lambda b: (b, 0, 0)),
        out_shape=jax.ShapeDtypeStruct((_B, 36, 1), jnp.float32),
        interpret=interpret,
    )(*args)
    z = out.reshape(_B, 2, 18)
    return jnp.pad(z, ((0, 0), (0, 27), (0, 0)))


# speedup vs baseline: 1.8115x; 1.0147x over previous
import jax
import jax.numpy as jnp
from jax.experimental import pallas as pl

_B, _N = 8, 1024
_D = 6
_E = 3
_K = 6


def _one(xb, ct, We1T_ref, be1_ref, We2T_ref, be2_ref, WgT_ref, bg_ref,
         Wc1T_ref, bc1_ref, Wc2T_ref, bc2_ref, ln_g_ref, ln_b_ref,
         Wn1T_ref, bn1_ref, Wn2T_ref, bn2_ref, Wm1T_ref, bm1_ref, Wm2T_ref, bm2_ref):
    N = _N
    coors = xb[:, _D:_D + _E]
    fT = ct[:_D]
    cT = ct[_D:_D + _E]

    dx = coors[:, 0:1] - ct[_D + 0:_D + 1, :]
    dy = coors[:, 1:2] - ct[_D + 1:_D + 2, :]
    dz = coors[:, 2:3] - ct[_D + 2:_D + 3, :]
    dist = dx * dx + dy * dy + dz * dz

    ct_hi = ct.astype(jnp.bfloat16).astype(jnp.float32)
    r1 = ct - ct_hi
    ct_md = r1.astype(jnp.bfloat16).astype(jnp.float32)
    ct_lo = r1 - ct_md
    gTs = [ct]
    rds = [jnp.zeros((1, N), jnp.float32)]
    m = jnp.zeros((1, N), jnp.float32)
    for _ in range(_K - 1):
        m = jnp.min(jnp.where(dist > m, dist, jnp.inf),
                    axis=0, keepdims=True)
        oh = jnp.where(dist == m, 1.0, 0.0)
        gT = ((jnp.dot(ct_lo, oh, preferred_element_type=jnp.float32)
               + jnp.dot(ct_md, oh, preferred_element_type=jnp.float32))
              + jnp.dot(ct_hi, oh, preferred_element_type=jnp.float32))
        gTs.append(gT)
        rds.append(m)

    fjT = jnp.concatenate([g[:_D] for g in gTs], axis=1)
    cjT = jnp.concatenate([g[_D:_D + _E] for g in gTs], axis=1)
    rdT = jnp.concatenate(rds, axis=1)
    fiT = jnp.concatenate([fT] * _K, axis=1)
    ciT = jnp.concatenate([cT] * _K, axis=1)
    relT = ciT - cjT

    edge_inT = jnp.concatenate([fiT, fjT, rdT], axis=0)
    hT = jax.nn.silu(jnp.dot(We1T_ref[...], edge_inT,
                             preferred_element_type=jnp.float32) + be1_ref[...])
    m_ijT = jax.nn.silu(jnp.dot(We2T_ref[...], hT,
                                preferred_element_type=jnp.float32) + be2_ref[...])
    gateT = jax.nn.sigmoid(jnp.dot(WgT_ref[...], m_ijT,
                                   preferred_element_type=jnp.float32) + bg_ref[...])
    m_ijT = m_ijT * gateT
    cwT = (jnp.dot(Wc2T_ref[...],
                   jax.nn.silu(jnp.dot(Wc1T_ref[...], m_ijT,
                                       preferred_element_type=jnp.float32) + bc1_ref[...]),
                   preferred_element_type=jnp.float32) + bc2_ref[...])
    cwT = jnp.clip(cwT, -1.0, 1.0)

    normT = jnp.sqrt(relT[0:1] ** 2 + relT[1:2] ** 2 + relT[2:3] ** 2)
    relnT = relT / jnp.maximum(normT, 1e-8)
    contribT = cwT * relnT
    coorsoT = cT + sum(contribT[:, k * N:(k + 1) * N] for k in range(_K))
    m_iT = sum(m_ijT[:, k * N:(k + 1) * N] for k in range(_K))

    mu = fT.mean(axis=0, keepdims=True)
    var = ((fT - mu) ** 2).mean(axis=0, keepdims=True)
    nfT = (fT - mu) / jnp.sqrt(var + 1e-5) * ln_g_ref[...] + ln_b_ref[...]
    node_inT = jnp.concatenate([nfT, m_iT], axis=0)
    nodeoT = (jnp.dot(Wn2T_ref[...],
                      jax.nn.silu(jnp.dot(Wn1T_ref[...], node_inT,
                                          preferred_element_type=jnp.float32) + bn1_ref[...]),
                      preferred_element_type=jnp.float32)
              + bn2_ref[...] + fT)

    zT = jnp.concatenate([nodeoT, coorsoT], axis=0)
    zmT = jnp.mean(zT, axis=1, keepdims=True)
    zz = (jnp.dot(Wm2T_ref[...],
                  jax.nn.relu(jnp.dot(Wm1T_ref[...], zmT,
                                      preferred_element_type=jnp.float32) + bm1_ref[...]),
                  preferred_element_type=jnp.float32) + bm2_ref[...])
    return zz


def _body(x_ref, xT_ref, *refs):
    *w_refs, out_ref = refs
    out_ref[0] = _one(x_ref[0], xT_ref[0], *w_refs)
    out_ref[1] = _one(x_ref[1], xT_ref[1], *w_refs)


def kernel(x, We1, be1, We2, be2, Wg, bg, Wc1, bc1, Wc2, bc2, ln_g, ln_b,
           Wn1, bn1, Wn2, bn2, Wm1, bm1, Wm2, bm2, interpret=False):
    xT = jnp.swapaxes(x, 1, 2)
    col = lambda a: a.reshape(-1, 1)
    full = lambda shp: pl.BlockSpec(shp, lambda b: (0,) * len(shp))
    wspec = lambda a: full(a.shape)
    args = [x, xT,
            We1.T, col(be1), We2.T, col(be2), Wg.T, col(bg),
            Wc1.T, col(bc1), Wc2.T, col(bc2), col(ln_g), col(ln_b),
            Wn1.T, col(bn1), Wn2.T, col(bn2), Wm1.T, col(bm1), Wm2.T, col(bm2)]
    out = pl.pallas_call(
        _body,
        grid=(_B // 2,),
        in_specs=[pl.BlockSpec((2, _N, _D + _E), lambda b: (b, 0, 0)),
                  pl.BlockSpec((2, _D + _E, _N), lambda b: (b, 0, 0))]
                 + [wspec(a) for a in args[2:]],
        out_specs=pl.BlockSpec((2, 36, 1), lambda b: (b, 0, 0)),
        out_shape=jax.ShapeDtypeStruct((_B, 36, 1), jnp.float32),
        interpret=interpret,
    )(*args)
    z = out.reshape(_B, 2, 18)
    return jnp.pad(z, ((0, 0), (0, 27), (0, 0)))
